# Initial kernel scaffold; baseline (speedup 1.0000x reference)
import jax
import jax.numpy as jnp
from jax.experimental import pallas as pl


def kernel(feat_matrix, padding, table, c_idx):
    raise NotImplementedError("write your pallas kernel here")


# speedup vs baseline: 15.0138x; 15.0138x over previous
import functools

import jax
import jax.numpy as jnp
from jax import lax
from jax.experimental import pallas as pl
from jax.experimental.pallas import tpu as pltpu
from jax.experimental.pallas import tpu_sc as plsc

NUM_FEATURES = 100000
DIM = 32
B, L, G = 1024, 50, 26
BL = B * L
N = BL * G
NW = 32
POS_PER_W = BL // NW
CPOS = 64
CHUNKS = POS_PER_W // CPOS
NIDX = CPOS * G
IPG = 128
NGATHER = NIDX // IPG

_mesh = plsc.VectorSubcoreMesh(core_axis_name="c", subcore_axis_name="s")


@functools.partial(
    pl.kernel,
    mesh=_mesh,
    out_type=jax.ShapeDtypeStruct((N, DIM), jnp.float32),
    compiler_params=pltpu.CompilerParams(use_tc_tiling_on_sc=False),
    scratch_types=[
        pltpu.VMEM((NIDX,), jnp.int32),
        pltpu.VMEM((NIDX, DIM), jnp.float32),
        pltpu.VMEM((CPOS,), jnp.int32),
        pltpu.SemaphoreType.DMA,
    ],
)
def _emb_kernel(fm_hbm, pad_hbm, table_hbm, out_hbm, idx_v, rows_v, pad_v, sem):
    wid = lax.axis_index("s") * 2 + lax.axis_index("c")
    zeros16 = jnp.zeros((16,), jnp.float32)

    def chunk_body(ci, carry):
        pbase = wid * POS_PER_W + ci * CPOS
        ibase = pbase * G
        pltpu.sync_copy(fm_hbm.at[pl.ds(ibase, NIDX)], idx_v)
        pltpu.sync_copy(pad_hbm.at[pl.ds(pbase, CPOS)], pad_v)
        copies = []
        for j in range(NGATHER):
            copies.append(
                pltpu.async_copy(
                    table_hbm.at[idx_v.at[pl.ds(j * IPG, IPG)]],
                    rows_v.at[pl.ds(j * IPG, IPG)],
                    sem,
                )
            )
        for cp in copies:
            cp.wait()

        def grp_body(g, c):
            pv = pad_v[pl.ds(g * 16, 16)]
            gbase = g * (16 * G)
            for lane in range(16):
                @pl.when(pv[lane] != 0)
                def _zero():
                    for k in range(G):
                        r = gbase + lane * G + k
                        rows_v[r, pl.ds(0, 16)] = zeros16
                        rows_v[r, pl.ds(16, 16)] = zeros16
            return c

        lax.fori_loop(0, CPOS // 16, grp_body, 0)

        pltpu.sync_copy(rows_v, out_hbm.at[pl.ds(ibase, NIDX)])
        return carry

    lax.fori_loop(0, CHUNKS, chunk_body, 0)


def kernel(feat_matrix, padding, table, c_idx):
    fm = jnp.take(feat_matrix, c_idx, axis=2).reshape(N)
    padi = padding.reshape(BL).astype(jnp.int32)
    out = _emb_kernel(fm, padi, table)
    return out.reshape(B, L, G * DIM)
